# baseline (speedup 1.0000x reference)
import functools

import jax
import jax.numpy as jnp
from jax import lax
from jax.experimental import pallas as pl
from jax.experimental.pallas import tpu as pltpu
from jax.experimental.pallas import tpu_sc as plsc

NN = 10000
D = 128
E = 320000
NW = 32
CH = 128
EPW = 10240
NCHUNK = EPW // CH
EPAD = NW * EPW - E
NB = 4
NGROUP = NCHUNK // NB
NP = 10112
RPT = NP // 16

_mesh = functools.partial(
    plsc.VectorSubcoreMesh, core_axis_name="c", subcore_axis_name="s",
    num_cores=2, num_subcores=16)


def _deg_body(dst_hbm, zf_hbm, of_hbm, out_hbm, idx_v, ones_v, accum_sh):
  cid = lax.axis_index("c")
  sid = lax.axis_index("s")
  wid = cid * 16 + sid

  pltpu.sync_copy(of_hbm, ones_v)
  pltpu.sync_copy(zf_hbm.at[pl.ds(sid * RPT, RPT)],
                  accum_sh.at[pl.ds(sid * RPT, RPT)])
  plsc.subcore_barrier()

  base = wid * EPW

  def chunk(j, c):
    pltpu.sync_copy(dst_hbm.at[pl.ds(base + j * CH, CH)], idx_v)
    pltpu.sync_copy(ones_v, accum_sh.at[idx_v], add=True)
    return c
  lax.fori_loop(0, NCHUNK, chunk, 0)

  plsc.subcore_barrier()
  pltpu.sync_copy(accum_sh.at[pl.ds(sid * RPT, RPT)],
                  out_hbm.at[cid, pl.ds(sid * RPT, RPT)])


def _deg_call(dst_p, zf, of):
  return pl.kernel(
      _deg_body,
      out_type=jax.ShapeDtypeStruct((2, NP, D), jnp.float32),
      mesh=_mesh(),
      scratch_types=[
          pltpu.VMEM((CH,), jnp.int32),
          pltpu.VMEM((CH, D), jnp.float32),
          pltpu.VMEM_SHARED((NP, D), jnp.float32),
      ],
  )(dst_p, zf, of)


def _scat_body(src_hbm, dst_hbm, g_hbm, zf_hbm, out_hbm,
               idxs_v, idxd_v, rows_v, accum_sh):
  cid = lax.axis_index("c")
  sid = lax.axis_index("s")
  wid = cid * 16 + sid

  pltpu.sync_copy(zf_hbm.at[pl.ds(sid * RPT, RPT)],
                  accum_sh.at[pl.ds(sid * RPT, RPT)])
  plsc.subcore_barrier()

  base = wid * EPW

  def chunk(j, c):
    pltpu.sync_copy(src_hbm.at[pl.ds(base + j * CH, CH)], idxs_v)
    pltpu.sync_copy(g_hbm.at[idxs_v], rows_v)
    pltpu.sync_copy(dst_hbm.at[pl.ds(base + j * CH, CH)], idxd_v)
    pltpu.sync_copy(rows_v, accum_sh.at[idxd_v], add=True)
    return c
  lax.fori_loop(0, NCHUNK, chunk, 0)

  plsc.subcore_barrier()
  pltpu.sync_copy(accum_sh.at[pl.ds(sid * RPT, RPT)],
                  out_hbm.at[cid, pl.ds(sid * RPT, RPT)])


def _scat_call(src_p, dst_p, g, zf):
  return pl.kernel(
      _scat_body,
      out_type=jax.ShapeDtypeStruct((2, NP, D), jnp.float32),
      mesh=_mesh(),
      scratch_types=[
          pltpu.VMEM((CH,), jnp.int32),
          pltpu.VMEM((CH,), jnp.int32),
          pltpu.VMEM((CH, D), jnp.float32),
          pltpu.VMEM_SHARED((NP, D), jnp.float32),
      ],
  )(src_p, dst_p, g, zf)


BR = 1000
_P = jax.lax.Precision.HIGHEST


def _tca_body(deg_ref, x_ref, norm_ref, g_ref):
  d = deg_ref[0] + deg_ref[1]
  nrm = lax.rsqrt(jnp.maximum(d[:, 0:1], 1.0))
  norm_ref[...] = nrm
  g_ref[...] = x_ref[...] * nrm


def _tca_call(deg16, x):
  return pl.pallas_call(
      _tca_body,
      grid=(NN // BR,),
      in_specs=[
          pl.BlockSpec((2, BR, D), lambda r: (0, r, 0)),
          pl.BlockSpec((BR, D), lambda r: (r, 0)),
      ],
      out_specs=[
          pl.BlockSpec((BR, 1), lambda r: (r, 0)),
          pl.BlockSpec((BR, D), lambda r: (r, 0)),
      ],
      out_shape=[
          jax.ShapeDtypeStruct((NN, 1), jnp.float32),
          jax.ShapeDtypeStruct((NN, D), jnp.float32),
      ],
  )(deg16, x)


def _tcb_body(s_ref, norm_ref, w_ref, b_ref, g2_ref):
  nrm = norm_ref[...]
  agg = (s_ref[0] + s_ref[1]) * nrm
  h = jnp.dot(agg, w_ref[...], preferred_element_type=jnp.float32,
              precision=_P) + b_ref[...]
  g2_ref[...] = jnp.maximum(h, 0.0) * nrm


def _tcb_call(s1, norm, w1, b1):
  return pl.pallas_call(
      _tcb_body,
      grid=(NN // BR,),
      in_specs=[
          pl.BlockSpec((2, BR, D), lambda r: (0, r, 0)),
          pl.BlockSpec((BR, 1), lambda r: (r, 0)),
          pl.BlockSpec((D, D), lambda r: (0, 0)),
          pl.BlockSpec((D,), lambda r: (0,)),
      ],
      out_specs=pl.BlockSpec((BR, D), lambda r: (r, 0)),
      out_shape=jax.ShapeDtypeStruct((NN, D), jnp.float32),
  )(s1, norm, w1, b1)


def _tcc_body(s_ref, norm_ref, w2_ref, b2_ref, wf1_ref, bf1_ref,
              wf2_ref, bf2_ref, out_ref):
  agg = (s_ref[0] + s_ref[1]) * norm_ref[...]
  h2 = jnp.maximum(
      jnp.dot(agg, w2_ref[...], preferred_element_type=jnp.float32,
              precision=_P) + b2_ref[...], 0.0)
  h3 = jnp.maximum(
      jnp.dot(h2, wf1_ref[...], preferred_element_type=jnp.float32,
              precision=_P) + bf1_ref[...], 0.0)
  out_ref[...] = jnp.dot(h3, wf2_ref[...], preferred_element_type=jnp.float32,
                         precision=_P) + bf2_ref[...]


def _tcc_call(s2, norm, w2, b2, wf1, bf1, wf2, bf2):
  nout = wf2.shape[1]
  return pl.pallas_call(
      _tcc_body,
      grid=(NN // BR,),
      in_specs=[
          pl.BlockSpec((2, BR, D), lambda r: (0, r, 0)),
          pl.BlockSpec((BR, 1), lambda r: (r, 0)),
          pl.BlockSpec((D, D), lambda r: (0, 0)),
          pl.BlockSpec((D,), lambda r: (0,)),
          pl.BlockSpec((D, D), lambda r: (0, 0)),
          pl.BlockSpec((D,), lambda r: (0,)),
          pl.BlockSpec((D, nout), lambda r: (0, 0)),
          pl.BlockSpec((nout,), lambda r: (0,)),
      ],
      out_specs=pl.BlockSpec((BR, nout), lambda r: (r, 0)),
      out_shape=jax.ShapeDtypeStruct((NN, nout), jnp.float32),
  )(s2, norm, w2, b2, wf1, bf1, wf2, bf2)


@jax.jit
def kernel(x, edge_index, W1, b1, W2, b2, Wf1, bf1, Wf2, bf2):
  src = edge_index[0]
  dst = edge_index[1]
  src_p = jnp.concatenate([src, jnp.zeros((EPAD,), jnp.int32)])
  dst_p = jnp.concatenate([dst, jnp.full((EPAD,), NN, jnp.int32)])
  zf = jnp.zeros((NP, D), jnp.float32)
  of = jnp.ones((CH, D), jnp.float32)

  deg_s = _deg_call(dst_p, zf, of)
  norm, g1 = _tca_call(deg_s, x)
  s1 = _scat_call(src_p, dst_p, g1, zf)
  g2 = _tcb_call(s1, norm, W1, b1)
  s2 = _scat_call(src_p, dst_p, g2, zf)
  return _tcc_call(s2, norm, W2, b2, Wf1, bf1, Wf2, bf2)


# speedup vs baseline: 1.1827x; 1.1827x over previous
import functools

import jax
import jax.numpy as jnp
from jax import lax
from jax.experimental import pallas as pl
from jax.experimental.pallas import tpu as pltpu
from jax.experimental.pallas import tpu_sc as plsc

NN = 10000
D = 128
E = 320000
NW = 32
CH = 128
EPW = 10240
NCHUNK = EPW // CH
EPAD = NW * EPW - E
NB = 2
NGROUP = NCHUNK // NB
NP = 10112
RPT = NP // 16

_mesh = functools.partial(
    plsc.VectorSubcoreMesh, core_axis_name="c", subcore_axis_name="s",
    num_cores=2, num_subcores=16)


def _deg_body(dst_hbm, zf_hbm, of_hbm, out_hbm, idx_v, ones_v, accum_sh):
  cid = lax.axis_index("c")
  sid = lax.axis_index("s")
  wid = cid * 16 + sid

  pltpu.sync_copy(of_hbm, ones_v)
  pltpu.sync_copy(zf_hbm.at[pl.ds(sid * RPT, RPT)],
                  accum_sh.at[pl.ds(sid * RPT, RPT)])
  plsc.subcore_barrier()

  base = wid * EPW

  def chunk(j, c):
    pltpu.sync_copy(dst_hbm.at[pl.ds(base + j * CH, CH)], idx_v)
    pltpu.sync_copy(ones_v, accum_sh.at[idx_v], add=True)
    return c
  lax.fori_loop(0, NCHUNK, chunk, 0)

  plsc.subcore_barrier()
  pltpu.sync_copy(accum_sh.at[pl.ds(sid * RPT, RPT)],
                  out_hbm.at[cid, pl.ds(sid * RPT, RPT)])


def _deg_call(dst_p, zf, of):
  return pl.kernel(
      _deg_body,
      out_type=jax.ShapeDtypeStruct((2, NP, D), jnp.float32),
      mesh=_mesh(),
      scratch_types=[
          pltpu.VMEM((CH,), jnp.int32),
          pltpu.VMEM((CH, D), jnp.float32),
          pltpu.VMEM_SHARED((NP, D), jnp.float32),
      ],
  )(dst_p, zf, of)


def _scat_body(src_hbm, dst_hbm, g_hbm, zf_hbm, out_hbm, *scr):
  idxs_v = scr[0:NB]
  rows_v = scr[NB:2 * NB]
  sems = scr[2 * NB:3 * NB]
  idxd_v = scr[3 * NB]
  accum_sh = scr[3 * NB + 1]

  cid = lax.axis_index("c")
  sid = lax.axis_index("s")
  wid = cid * 16 + sid

  pltpu.sync_copy(zf_hbm.at[pl.ds(sid * RPT, RPT)],
                  accum_sh.at[pl.ds(sid * RPT, RPT)])
  plsc.subcore_barrier()

  base = wid * EPW

  for b in range(NB):
    pltpu.sync_copy(src_hbm.at[pl.ds(base + b * CH, CH)], idxs_v[b])
    pltpu.async_copy(g_hbm.at[idxs_v[b]], rows_v[b], sems[b])

  def group(g, c):
    for b in range(NB):
      j = g * NB + b
      pltpu.make_async_copy(g_hbm.at[idxs_v[b]], rows_v[b], sems[b]).wait()
      pltpu.sync_copy(dst_hbm.at[pl.ds(base + j * CH, CH)], idxd_v)
      pltpu.sync_copy(rows_v[b], accum_sh.at[idxd_v], add=True)
      jn = jnp.minimum(j + NB, NCHUNK - 1)
      pltpu.sync_copy(src_hbm.at[pl.ds(base + jn * CH, CH)], idxs_v[b])
      pltpu.async_copy(g_hbm.at[idxs_v[b]], rows_v[b], sems[b])
    return c
  lax.fori_loop(0, NGROUP, group, 0)

  for b in range(NB):
    pltpu.make_async_copy(g_hbm.at[idxs_v[b]], rows_v[b], sems[b]).wait()

  plsc.subcore_barrier()
  pltpu.sync_copy(accum_sh.at[pl.ds(sid * RPT, RPT)],
                  out_hbm.at[cid, pl.ds(sid * RPT, RPT)])


def _scat_call(src_p, dst_p, g, zf):
  return pl.kernel(
      _scat_body,
      out_type=jax.ShapeDtypeStruct((2, NP, D), jnp.float32),
      mesh=_mesh(),
      scratch_types=(
          [pltpu.VMEM((CH,), jnp.int32) for _ in range(NB)]
          + [pltpu.VMEM((CH, D), jnp.float32) for _ in range(NB)]
          + [pltpu.SemaphoreType.DMA for _ in range(NB)]
          + [
              pltpu.VMEM((CH,), jnp.int32),
              pltpu.VMEM_SHARED((NP, D), jnp.float32),
          ]
      ),
  )(src_p, dst_p, g, zf)


BR = 1000
_P = jax.lax.Precision.HIGHEST


def _tca_body(deg_ref, x_ref, norm_ref, g_ref):
  d = deg_ref[0] + deg_ref[1]
  nrm = lax.rsqrt(jnp.maximum(d[:, 0:1], 1.0))
  norm_ref[...] = nrm
  g_ref[...] = x_ref[...] * nrm


def _tca_call(deg16, x):
  return pl.pallas_call(
      _tca_body,
      grid=(NN // BR,),
      in_specs=[
          pl.BlockSpec((2, BR, D), lambda r: (0, r, 0)),
          pl.BlockSpec((BR, D), lambda r: (r, 0)),
      ],
      out_specs=[
          pl.BlockSpec((BR, 1), lambda r: (r, 0)),
          pl.BlockSpec((BR, D), lambda r: (r, 0)),
      ],
      out_shape=[
          jax.ShapeDtypeStruct((NN, 1), jnp.float32),
          jax.ShapeDtypeStruct((NN, D), jnp.float32),
      ],
  )(deg16, x)


def _tcb_body(s_ref, norm_ref, w_ref, b_ref, g2_ref):
  nrm = norm_ref[...]
  agg = (s_ref[0] + s_ref[1]) * nrm
  h = jnp.dot(agg, w_ref[...], preferred_element_type=jnp.float32,
              precision=_P) + b_ref[...]
  g2_ref[...] = jnp.maximum(h, 0.0) * nrm


def _tcb_call(s1, norm, w1, b1):
  return pl.pallas_call(
      _tcb_body,
      grid=(NN // BR,),
      in_specs=[
          pl.BlockSpec((2, BR, D), lambda r: (0, r, 0)),
          pl.BlockSpec((BR, 1), lambda r: (r, 0)),
          pl.BlockSpec((D, D), lambda r: (0, 0)),
          pl.BlockSpec((D,), lambda r: (0,)),
      ],
      out_specs=pl.BlockSpec((BR, D), lambda r: (r, 0)),
      out_shape=jax.ShapeDtypeStruct((NN, D), jnp.float32),
  )(s1, norm, w1, b1)


def _tcc_body(s_ref, norm_ref, w2_ref, b2_ref, wf1_ref, bf1_ref,
              wf2_ref, bf2_ref, out_ref):
  agg = (s_ref[0] + s_ref[1]) * norm_ref[...]
  h2 = jnp.maximum(
      jnp.dot(agg, w2_ref[...], preferred_element_type=jnp.float32,
              precision=_P) + b2_ref[...], 0.0)
  h3 = jnp.maximum(
      jnp.dot(h2, wf1_ref[...], preferred_element_type=jnp.float32,
              precision=_P) + bf1_ref[...], 0.0)
  out_ref[...] = jnp.dot(h3, wf2_ref[...], preferred_element_type=jnp.float32,
                         precision=_P) + bf2_ref[...]


def _tcc_call(s2, norm, w2, b2, wf1, bf1, wf2, bf2):
  nout = wf2.shape[1]
  return pl.pallas_call(
      _tcc_body,
      grid=(NN // BR,),
      in_specs=[
          pl.BlockSpec((2, BR, D), lambda r: (0, r, 0)),
          pl.BlockSpec((BR, 1), lambda r: (r, 0)),
          pl.BlockSpec((D, D), lambda r: (0, 0)),
          pl.BlockSpec((D,), lambda r: (0,)),
          pl.BlockSpec((D, D), lambda r: (0, 0)),
          pl.BlockSpec((D,), lambda r: (0,)),
          pl.BlockSpec((D, nout), lambda r: (0, 0)),
          pl.BlockSpec((nout,), lambda r: (0,)),
      ],
      out_specs=pl.BlockSpec((BR, nout), lambda r: (r, 0)),
      out_shape=jax.ShapeDtypeStruct((NN, nout), jnp.float32),
  )(s2, norm, w2, b2, wf1, bf1, wf2, bf2)


@jax.jit
def kernel(x, edge_index, W1, b1, W2, b2, Wf1, bf1, Wf2, bf2):
  src = edge_index[0]
  dst = edge_index[1]
  src_p = jnp.concatenate([src, jnp.zeros((EPAD,), jnp.int32)])
  dst_p = jnp.concatenate([dst, jnp.full((EPAD,), NN, jnp.int32)])
  zf = jnp.zeros((NP, D), jnp.float32)
  of = jnp.ones((CH, D), jnp.float32)

  deg_s = _deg_call(dst_p, zf, of)
  norm, g1 = _tca_call(deg_s, x)
  s1 = _scat_call(src_p, dst_p, g1, zf)
  g2 = _tcb_call(s1, norm, W1, b1)
  s2 = _scat_call(src_p, dst_p, g2, zf)
  return _tcc_call(s2, norm, W2, b2, Wf1, bf1, Wf2, bf2)


# speedup vs baseline: 2.7485x; 2.3238x over previous
import functools

import jax
import jax.numpy as jnp
from jax import lax
from jax.experimental import pallas as pl
from jax.experimental.pallas import tpu as pltpu
from jax.experimental.pallas import tpu_sc as plsc

NN = 10000
D = 128
E = 320000
NW = 32
CH = 128
EPW = 10240
NCHUNK = EPW // CH
EPAD = NW * EPW - E
NB = 2
NGROUP = NCHUNK // NB
NP = 10112
RPT = NP // 16

_mesh = functools.partial(
    plsc.VectorSubcoreMesh, core_axis_name="c", subcore_axis_name="s",
    num_cores=2, num_subcores=16)


def _deg_body(dst_hbm, zf_hbm, of_hbm, out_hbm, idx_v, ones_v, accum_sh):
  cid = lax.axis_index("c")
  sid = lax.axis_index("s")
  wid = cid * 16 + sid

  pltpu.sync_copy(of_hbm, ones_v)
  pltpu.sync_copy(zf_hbm.at[pl.ds(sid * RPT, RPT)],
                  accum_sh.at[pl.ds(sid * RPT, RPT)])
  plsc.subcore_barrier()

  base = wid * EPW

  def chunk(j, c):
    pltpu.sync_copy(dst_hbm.at[pl.ds(base + j * CH, CH)], idx_v)
    pltpu.sync_copy(ones_v, accum_sh.at[idx_v], add=True)
    return c
  lax.fori_loop(0, NCHUNK, chunk, 0)

  plsc.subcore_barrier()
  pltpu.sync_copy(accum_sh.at[pl.ds(sid * RPT, RPT)],
                  out_hbm.at[cid, pl.ds(sid * RPT, RPT)])


def _deg_call(dst_p, zf, of):
  return pl.kernel(
      _deg_body,
      out_type=jax.ShapeDtypeStruct((2, NP, D), jnp.float32),
      mesh=_mesh(),
      scratch_types=[
          pltpu.VMEM((CH,), jnp.int32),
          pltpu.VMEM((CH, D), jnp.float32),
          pltpu.VMEM_SHARED((NP, D), jnp.float32),
      ],
  )(dst_p, zf, of)


def _scat_body(src_hbm, dst_hbm, g_hbm, zf_hbm, out_hbm, *scr):
  idxs_v = scr[0:NB]
  rows_v = scr[NB:2 * NB]
  sems = scr[2 * NB:3 * NB]
  idxd_v = scr[3 * NB]
  accum_sh = scr[3 * NB + 1]

  cid = lax.axis_index("c")
  sid = lax.axis_index("s")
  wid = cid * 16 + sid

  pltpu.sync_copy(zf_hbm.at[pl.ds(sid * RPT, RPT)],
                  accum_sh.at[pl.ds(sid * RPT, RPT)])
  plsc.subcore_barrier()

  base = wid * EPW

  for b in range(NB):
    pltpu.sync_copy(src_hbm.at[pl.ds(base + b * CH, CH)], idxs_v[b])
    pltpu.async_copy(g_hbm.at[idxs_v[b]], rows_v[b], sems[b])

  def group(g, c):
    for b in range(NB):
      j = g * NB + b
      pltpu.make_async_copy(g_hbm.at[idxs_v[b]], rows_v[b], sems[b]).wait()
      pltpu.sync_copy(dst_hbm.at[pl.ds(base + j * CH, CH)], idxd_v)
      pltpu.sync_copy(rows_v[b], accum_sh.at[idxd_v], add=True)
      jn = jnp.minimum(j + NB, NCHUNK - 1)
      pltpu.sync_copy(src_hbm.at[pl.ds(base + jn * CH, CH)], idxs_v[b])
      pltpu.async_copy(g_hbm.at[idxs_v[b]], rows_v[b], sems[b])
    return c
  lax.fori_loop(0, NGROUP, group, 0)

  for b in range(NB):
    pltpu.make_async_copy(g_hbm.at[idxs_v[b]], rows_v[b], sems[b]).wait()

  plsc.subcore_barrier()
  pltpu.sync_copy(accum_sh.at[pl.ds(sid * RPT, RPT)],
                  out_hbm.at[cid, pl.ds(sid * RPT, RPT)])


def _scat_call(src_p, dst_p, g, zf):
  return pl.kernel(
      _scat_body,
      out_type=jax.ShapeDtypeStruct((2, NP, D), jnp.float32),
      mesh=_mesh(),
      scratch_types=(
          [pltpu.VMEM((CH,), jnp.int32) for _ in range(NB)]
          + [pltpu.VMEM((CH, D), jnp.float32) for _ in range(NB)]
          + [pltpu.SemaphoreType.DMA for _ in range(NB)]
          + [
              pltpu.VMEM((CH,), jnp.int32),
              pltpu.VMEM_SHARED((NP, D), jnp.float32),
          ]
      ),
  )(src_p, dst_p, g, zf)


BR = 1000
_P = jax.lax.Precision.HIGHEST


def _tca_body(deg_ref, x_ref, norm_ref, g_ref):
  d = deg_ref[0] + deg_ref[1]
  nrm = lax.rsqrt(jnp.maximum(d[:, 0:1], 1.0))
  norm_ref[...] = nrm
  g_ref[...] = x_ref[...] * nrm


def _tca_call(deg16, x):
  return pl.pallas_call(
      _tca_body,
      grid=(NN // BR,),
      in_specs=[
          pl.BlockSpec((2, BR, D), lambda r: (0, r, 0)),
          pl.BlockSpec((BR, D), lambda r: (r, 0)),
      ],
      out_specs=[
          pl.BlockSpec((BR, 1), lambda r: (r, 0)),
          pl.BlockSpec((BR, D), lambda r: (r, 0)),
      ],
      out_shape=[
          jax.ShapeDtypeStruct((NN, 1), jnp.float32),
          jax.ShapeDtypeStruct((NN, D), jnp.float32),
      ],
  )(deg16, x)


def _tcb_body(s_ref, norm_ref, w_ref, b_ref, g2_ref):
  nrm = norm_ref[...]
  agg = (s_ref[0] + s_ref[1]) * nrm
  h = jnp.dot(agg, w_ref[...], preferred_element_type=jnp.float32,
              precision=_P) + b_ref[...]
  g2_ref[...] = jnp.maximum(h, 0.0) * nrm


def _tcb_call(s1, norm, w1, b1):
  return pl.pallas_call(
      _tcb_body,
      grid=(NN // BR,),
      in_specs=[
          pl.BlockSpec((2, BR, D), lambda r: (0, r, 0)),
          pl.BlockSpec((BR, 1), lambda r: (r, 0)),
          pl.BlockSpec((D, D), lambda r: (0, 0)),
          pl.BlockSpec((D,), lambda r: (0,)),
      ],
      out_specs=pl.BlockSpec((BR, D), lambda r: (r, 0)),
      out_shape=jax.ShapeDtypeStruct((NN, D), jnp.float32),
  )(s1, norm, w1, b1)


def _tcc_body(s_ref, norm_ref, w2_ref, b2_ref, wf1_ref, bf1_ref,
              wf2_ref, bf2_ref, out_ref):
  agg = (s_ref[0] + s_ref[1]) * norm_ref[...]
  h2 = jnp.maximum(
      jnp.dot(agg, w2_ref[...], preferred_element_type=jnp.float32,
              precision=_P) + b2_ref[...], 0.0)
  h3 = jnp.maximum(
      jnp.dot(h2, wf1_ref[...], preferred_element_type=jnp.float32,
              precision=_P) + bf1_ref[...], 0.0)
  out_ref[...] = jnp.dot(h3, wf2_ref[...], preferred_element_type=jnp.float32,
                         precision=_P) + bf2_ref[...]


def _tcc_call(s2, norm, w2, b2, wf1, bf1, wf2, bf2):
  nout = wf2.shape[1]
  return pl.pallas_call(
      _tcc_body,
      grid=(NN // BR,),
      in_specs=[
          pl.BlockSpec((2, BR, D), lambda r: (0, r, 0)),
          pl.BlockSpec((BR, 1), lambda r: (r, 0)),
          pl.BlockSpec((D, D), lambda r: (0, 0)),
          pl.BlockSpec((D,), lambda r: (0,)),
          pl.BlockSpec((D, D), lambda r: (0, 0)),
          pl.BlockSpec((D,), lambda r: (0,)),
          pl.BlockSpec((D, nout), lambda r: (0, 0)),
          pl.BlockSpec((nout,), lambda r: (0,)),
      ],
      out_specs=pl.BlockSpec((BR, nout), lambda r: (r, 0)),
      out_shape=jax.ShapeDtypeStruct((NN, nout), jnp.float32),
  )(s2, norm, w2, b2, wf1, bf1, wf2, bf2)


@jax.jit
def kernel(x, edge_index, W1, b1, W2, b2, Wf1, bf1, Wf2, bf2):
  src = edge_index[0]
  dst = edge_index[1]
  pad = jnp.arange(EPAD, dtype=jnp.int32)
  src_p = jnp.concatenate([src, pad % NN])
  dst_p = jnp.concatenate([dst, NN + pad % (NP - NN)])
  zf = jnp.zeros((NP, D), jnp.float32)
  of = jnp.ones((CH, D), jnp.float32)

  deg_s = _deg_call(dst_p, zf, of)
  norm, g1 = _tca_call(deg_s, x)
  s1 = _scat_call(src_p, dst_p, g1, zf)
  g2 = _tcb_call(s1, norm, W1, b1)
  s2 = _scat_call(src_p, dst_p, g2, zf)
  return _tcc_call(s2, norm, W2, b2, Wf1, bf1, Wf2, bf2)


# speedup vs baseline: 3.0622x; 1.1141x over previous
import functools

import jax
import jax.numpy as jnp
from jax import lax
from jax.experimental import pallas as pl
from jax.experimental.pallas import tpu as pltpu
from jax.experimental.pallas import tpu_sc as plsc

NN = 10000
D = 128
E = 320000
NW = 32
CH = 128
EPW = 10240
NCHUNK = EPW // CH
EPAD = NW * EPW - E
NB = 2
NGROUP = NCHUNK // NB
NP = 10112
RPT = NP // 16

_mesh = functools.partial(
    plsc.VectorSubcoreMesh, core_axis_name="c", subcore_axis_name="s",
    num_cores=2, num_subcores=16)


def _deg_body(dst_hbm, zf_hbm, of_hbm, out_hbm, idx_v, ones_v, accum_sh):
  cid = lax.axis_index("c")
  sid = lax.axis_index("s")
  wid = cid * 16 + sid

  pltpu.sync_copy(of_hbm, ones_v)
  pltpu.sync_copy(zf_hbm.at[pl.ds(sid * RPT, RPT)],
                  accum_sh.at[pl.ds(sid * RPT, RPT)])
  plsc.subcore_barrier()

  base = wid * EPW

  def chunk(j, c):
    pltpu.sync_copy(dst_hbm.at[pl.ds(base + j * CH, CH)], idx_v)
    pltpu.sync_copy(ones_v, accum_sh.at[idx_v], add=True)
    return c
  lax.fori_loop(0, NCHUNK, chunk, 0)

  plsc.subcore_barrier()
  pltpu.sync_copy(accum_sh.at[pl.ds(sid * RPT, RPT)],
                  out_hbm.at[cid, pl.ds(sid * RPT, RPT)])


def _deg_call(dst_p, zf, of):
  return pl.kernel(
      _deg_body,
      out_type=jax.ShapeDtypeStruct((2, NP, D), jnp.float32),
      mesh=_mesh(),
      scratch_types=[
          pltpu.VMEM((CH,), jnp.int32),
          pltpu.VMEM((CH, D), jnp.float32),
          pltpu.VMEM_SHARED((NP, D), jnp.float32),
      ],
  )(dst_p, zf, of)


def _scat_body(src_hbm, dst_hbm, g_hbm, zf_hbm, out_hbm, *scr):
  idxs_v = scr[0:NB]
  rows_v = scr[NB:2 * NB]
  sems = scr[2 * NB:3 * NB]
  idxd_v = scr[3 * NB:4 * NB]
  sems_d = scr[4 * NB:5 * NB]
  accum_sh = scr[5 * NB]

  cid = lax.axis_index("c")
  sid = lax.axis_index("s")
  wid = cid * 16 + sid

  pltpu.sync_copy(zf_hbm.at[pl.ds(sid * RPT, RPT)],
                  accum_sh.at[pl.ds(sid * RPT, RPT)])
  plsc.subcore_barrier()

  base = wid * EPW

  for b in range(NB):
    pltpu.sync_copy(src_hbm.at[pl.ds(base + b * CH, CH)], idxs_v[b])
    pltpu.async_copy(g_hbm.at[idxs_v[b]], rows_v[b], sems[b])
    pltpu.async_copy(dst_hbm.at[pl.ds(base + b * CH, CH)], idxd_v[b], sems_d[b])

  def group(g, c):
    for b in range(NB):
      j = g * NB + b
      pltpu.make_async_copy(g_hbm.at[idxs_v[b]], rows_v[b], sems[b]).wait()
      pltpu.make_async_copy(dst_hbm.at[pl.ds(base + j * CH, CH)], idxd_v[b],
                            sems_d[b]).wait()
      pltpu.sync_copy(rows_v[b], accum_sh.at[idxd_v[b]], add=True)
      jn = jnp.minimum(j + NB, NCHUNK - 1)
      pltpu.sync_copy(src_hbm.at[pl.ds(base + jn * CH, CH)], idxs_v[b])
      pltpu.async_copy(g_hbm.at[idxs_v[b]], rows_v[b], sems[b])
      pltpu.async_copy(dst_hbm.at[pl.ds(base + jn * CH, CH)], idxd_v[b],
                       sems_d[b])
    return c
  lax.fori_loop(0, NGROUP, group, 0)

  for b in range(NB):
    pltpu.make_async_copy(g_hbm.at[idxs_v[b]], rows_v[b], sems[b]).wait()
    pltpu.make_async_copy(dst_hbm.at[pl.ds(0, CH)], idxd_v[b], sems_d[b]).wait()

  plsc.subcore_barrier()
  pltpu.sync_copy(accum_sh.at[pl.ds(sid * RPT, RPT)],
                  out_hbm.at[cid, pl.ds(sid * RPT, RPT)])


def _scat_call(src_p, dst_p, g, zf):
  return pl.kernel(
      _scat_body,
      out_type=jax.ShapeDtypeStruct((2, NP, D), jnp.float32),
      mesh=_mesh(),
      scratch_types=(
          [pltpu.VMEM((CH,), jnp.int32) for _ in range(NB)]
          + [pltpu.VMEM((CH, D), jnp.float32) for _ in range(NB)]
          + [pltpu.SemaphoreType.DMA for _ in range(NB)]
          + [pltpu.VMEM((CH,), jnp.int32) for _ in range(NB)]
          + [pltpu.SemaphoreType.DMA for _ in range(NB)]
          + [pltpu.VMEM_SHARED((NP, D), jnp.float32)]
      ),
  )(src_p, dst_p, g, zf)


BR = 1000
_P = jax.lax.Precision.HIGHEST


def _tca_body(deg_ref, x_ref, norm_ref, g_ref):
  d = deg_ref[0] + deg_ref[1]
  nrm = lax.rsqrt(jnp.maximum(d[:, 0:1], 1.0))
  norm_ref[...] = nrm
  g_ref[...] = x_ref[...] * nrm


def _tca_call(deg16, x):
  return pl.pallas_call(
      _tca_body,
      grid=(NN // BR,),
      in_specs=[
          pl.BlockSpec((2, BR, D), lambda r: (0, r, 0)),
          pl.BlockSpec((BR, D), lambda r: (r, 0)),
      ],
      out_specs=[
          pl.BlockSpec((BR, 1), lambda r: (r, 0)),
          pl.BlockSpec((BR, D), lambda r: (r, 0)),
      ],
      out_shape=[
          jax.ShapeDtypeStruct((NN, 1), jnp.float32),
          jax.ShapeDtypeStruct((NN, D), jnp.float32),
      ],
  )(deg16, x)


def _tcb_body(s_ref, norm_ref, w_ref, b_ref, g2_ref):
  nrm = norm_ref[...]
  agg = (s_ref[0] + s_ref[1]) * nrm
  h = jnp.dot(agg, w_ref[...], preferred_element_type=jnp.float32,
              precision=_P) + b_ref[...]
  g2_ref[...] = jnp.maximum(h, 0.0) * nrm


def _tcb_call(s1, norm, w1, b1):
  return pl.pallas_call(
      _tcb_body,
      grid=(NN // BR,),
      in_specs=[
          pl.BlockSpec((2, BR, D), lambda r: (0, r, 0)),
          pl.BlockSpec((BR, 1), lambda r: (r, 0)),
          pl.BlockSpec((D, D), lambda r: (0, 0)),
          pl.BlockSpec((D,), lambda r: (0,)),
      ],
      out_specs=pl.BlockSpec((BR, D), lambda r: (r, 0)),
      out_shape=jax.ShapeDtypeStruct((NN, D), jnp.float32),
  )(s1, norm, w1, b1)


def _tcc_body(s_ref, norm_ref, w2_ref, b2_ref, wf1_ref, bf1_ref,
              wf2_ref, bf2_ref, out_ref):
  agg = (s_ref[0] + s_ref[1]) * norm_ref[...]
  h2 = jnp.maximum(
      jnp.dot(agg, w2_ref[...], preferred_element_type=jnp.float32,
              precision=_P) + b2_ref[...], 0.0)
  h3 = jnp.maximum(
      jnp.dot(h2, wf1_ref[...], preferred_element_type=jnp.float32,
              precision=_P) + bf1_ref[...], 0.0)
  out_ref[...] = jnp.dot(h3, wf2_ref[...], preferred_element_type=jnp.float32,
                         precision=_P) + bf2_ref[...]


def _tcc_call(s2, norm, w2, b2, wf1, bf1, wf2, bf2):
  nout = wf2.shape[1]
  return pl.pallas_call(
      _tcc_body,
      grid=(NN // BR,),
      in_specs=[
          pl.BlockSpec((2, BR, D), lambda r: (0, r, 0)),
          pl.BlockSpec((BR, 1), lambda r: (r, 0)),
          pl.BlockSpec((D, D), lambda r: (0, 0)),
          pl.BlockSpec((D,), lambda r: (0,)),
          pl.BlockSpec((D, D), lambda r: (0, 0)),
          pl.BlockSpec((D,), lambda r: (0,)),
          pl.BlockSpec((D, nout), lambda r: (0, 0)),
          pl.BlockSpec((nout,), lambda r: (0,)),
      ],
      out_specs=pl.BlockSpec((BR, nout), lambda r: (r, 0)),
      out_shape=jax.ShapeDtypeStruct((NN, nout), jnp.float32),
  )(s2, norm, w2, b2, wf1, bf1, wf2, bf2)


@jax.jit
def kernel(x, edge_index, W1, b1, W2, b2, Wf1, bf1, Wf2, bf2):
  src = edge_index[0]
  dst = edge_index[1]
  pad = jnp.arange(EPAD, dtype=jnp.int32)
  src_p = jnp.concatenate([src, pad % NN])
  dst_p = jnp.concatenate([dst, NN + pad % (NP - NN)])
  zf = jnp.zeros((NP, D), jnp.float32)
  of = jnp.ones((CH, D), jnp.float32)

  deg_s = _deg_call(dst_p, zf, of)
  norm, g1 = _tca_call(deg_s, x)
  s1 = _scat_call(src_p, dst_p, g1, zf)
  g2 = _tcb_call(s1, norm, W1, b1)
  s2 = _scat_call(src_p, dst_p, g2, zf)
  return _tcc_call(s2, norm, W2, b2, Wf1, bf1, Wf2, bf2)
